# Initial kernel scaffold; baseline (speedup 1.0000x reference)
import jax
import jax.numpy as jnp
from jax.experimental import pallas as pl


def kernel(x, weight):
    raise NotImplementedError("write your pallas kernel here")


# speedup vs baseline: 1.3071x; 1.3071x over previous
import functools

import jax
import jax.numpy as jnp
from jax import lax
from jax.experimental import pallas as pl
from jax.experimental.pallas import tpu as pltpu
from jax.experimental.pallas import tpu_sc as plsc

NC = 2
NS = 16
NW = NC * NS
CHUNK = 128


def _make_gather(n_rows, emb_dim):
  n_chunks = n_rows // CHUNK
  chunks_per_w = n_chunks // NW
  rows_per_w = chunks_per_w * CHUNK

  mesh = plsc.VectorSubcoreMesh(core_axis_name="c", subcore_axis_name="s")

  @functools.partial(
      pl.kernel,
      out_type=jax.ShapeDtypeStruct((n_rows, emb_dim), jnp.float32),
      mesh=mesh,
      scratch_types=[
          pltpu.VMEM((chunks_per_w, CHUNK), jnp.int32),
          pltpu.VMEM((CHUNK, emb_dim), jnp.float32),
          pltpu.SemaphoreType.DMA,
      ],
      compiler_params=pltpu.CompilerParams(use_tc_tiling_on_sc=False),
  )
  def k(idx_hbm, w_hbm, out_hbm, idx_v, rows_v, sem):
    wid = lax.axis_index("s") * NC + lax.axis_index("c")
    crow0 = wid * chunks_per_w
    base = wid * rows_per_w
    pltpu.sync_copy(idx_hbm.at[pl.ds(crow0, chunks_per_w)], idx_v)

    @pl.loop(0, chunks_per_w)
    def _body(j):
      pltpu.async_copy(w_hbm.at[idx_v.at[j]], rows_v, sem).wait()
      pltpu.sync_copy(rows_v, out_hbm.at[pl.ds(base + j * CHUNK, CHUNK)])

  return k


def kernel(x, weight):
  b, l = x.shape
  emb_dim = weight.shape[1]
  idx = x.reshape(-1).astype(jnp.int32).reshape(-1, CHUNK)
  out = _make_gather(b * l, emb_dim)(idx, weight)
  return out.reshape(b, l, emb_dim)


# speedup vs baseline: 1.4993x; 1.1471x over previous
import functools

import jax
import jax.numpy as jnp
from jax import lax
from jax.experimental import pallas as pl
from jax.experimental.pallas import tpu as pltpu
from jax.experimental.pallas import tpu_sc as plsc

NC = 2
NS = 16
NW = NC * NS
CHUNK = 128
NBUF = 8
DEPTH = 4


def _make_gather(n_rows, emb_dim):
  n_chunks = n_rows // CHUNK
  chunks_per_w = n_chunks // NW
  rows_per_w = chunks_per_w * CHUNK
  assert chunks_per_w % NBUF == 0 and chunks_per_w > NBUF

  mesh = plsc.VectorSubcoreMesh(core_axis_name="c", subcore_axis_name="s")

  @functools.partial(
      pl.kernel,
      out_type=jax.ShapeDtypeStruct((n_rows, emb_dim), jnp.float32),
      mesh=mesh,
      scratch_types=[
          pltpu.VMEM((chunks_per_w, CHUNK), jnp.int32),
          [pltpu.VMEM((CHUNK, emb_dim), jnp.float32) for _ in range(NBUF)],
          [pltpu.SemaphoreType.DMA for _ in range(NBUF)],
          [pltpu.SemaphoreType.DMA for _ in range(NBUF)],
      ],
      compiler_params=pltpu.CompilerParams(use_tc_tiling_on_sc=False),
  )
  def k(idx_hbm, w_hbm, out_hbm, idx_v, bufs, gsems, ssems):
    wid = lax.axis_index("s") * NC + lax.axis_index("c")
    crow0 = wid * chunks_per_w
    base = wid * rows_per_w
    pltpu.sync_copy(idx_hbm.at[pl.ds(crow0, chunks_per_w)], idx_v)

    def fire_gather(c, slot):
      pltpu.async_copy(w_hbm.at[idx_v.at[c]], bufs[slot], gsems[slot])

    def wait_gather(slot):
      pltpu.make_async_copy(w_hbm.at[idx_v.at[0]], bufs[slot],
                            gsems[slot]).wait()

    def fire_store(c, slot):
      pltpu.async_copy(bufs[slot], out_hbm.at[pl.ds(base + c * CHUNK, CHUNK)],
                       ssems[slot])

    def wait_store(slot):
      pltpu.make_async_copy(bufs[slot], out_hbm.at[pl.ds(base, CHUNK)],
                            ssems[slot]).wait()

    for c in range(DEPTH):
      fire_gather(c, c % NBUF)
    for c in range(DEPTH, NBUF):
      fire_gather(c, c % NBUF)
      wait_gather((c - DEPTH) % NBUF)
      fire_store(c - DEPTH, (c - DEPTH) % NBUF)

    @pl.loop(NBUF, chunks_per_w, step=NBUF)
    def _body(j):
      for b in range(NBUF):
        c = j + b
        wait_store(b)
        fire_gather(c, b)
        bd = (b - DEPTH) % NBUF
        wait_gather(bd)
        fire_store(c - DEPTH, bd)

    n = chunks_per_w
    for c in range(n, n + DEPTH):
      b = c % NBUF
      wait_store(b)
      bd = (c - DEPTH) % NBUF
      wait_gather(bd)
      fire_store(c - DEPTH, bd)
    for c in range(n + DEPTH, n + NBUF):
      wait_store(c % NBUF)

  return k


def kernel(x, weight):
  b, l = x.shape
  emb_dim = weight.shape[1]
  idx = x.reshape(-1).astype(jnp.int32).reshape(-1, CHUNK)
  out = _make_gather(b * l, emb_dim)(idx, weight)
  return out.reshape(b, l, emb_dim)
